# Initial kernel scaffold; baseline (speedup 1.0000x reference)
import jax
import jax.numpy as jnp
from jax.experimental import pallas as pl


def kernel(query, value, reference_points, spatial_shapes, Wv, bv, Woff, boff, Wattn, battn, Wout, bout):
    raise NotImplementedError("write your pallas kernel here")


# speedup vs baseline: 61.6665x; 61.6665x over previous
import functools
import numpy as np
import jax
import jax.numpy as jnp
from jax import lax
from jax.experimental import pallas as pl
from jax.experimental.pallas import tpu as pltpu
from jax.experimental.pallas import tpu_sc as plsc

BS = 2
EMBED = 256
HEADS = 8
LEVELS = 4
POINTS = 4
_SHAPES = np.array([[64, 64], [32, 32], [16, 16], [8, 8]], dtype=np.int64)
_SIZES = (_SHAPES[:, 0] * _SHAPES[:, 1]).astype(np.int64)
NK = int(_SIZES.sum())
NQ = NK
HLP = HEADS * LEVELS * POINTS
ROWS = BS * NQ
RBLK = 544
NBLK = ROWS // RBLK
CQ = 160
NCHUNK = NQ // CQ

_lane_l = np.tile(np.repeat(np.arange(LEVELS), POINTS), HEADS)
_WL = _SHAPES[_lane_l, 1].astype(np.float32)
_HL = _SHAPES[_lane_l, 0].astype(np.float32)
_STARTS = np.concatenate([[0], np.cumsum(_SIZES)]).astype(np.float32)
_STARTL = _STARTS[_lane_l]

_CONSTS = np.zeros((8, HLP), np.float32)
_CONSTS[0] = _WL
_CONSTS[1] = _HL
_CONSTS[2] = _STARTL

_h = np.arange(HLP) // (LEVELS * POINTS)
_l = (np.arange(HLP) // POINTS) % LEVELS
_p = np.arange(HLP) % POINTS
_old_base = ((_h * LEVELS + _l) * POINTS + _p) * 2
_OFF_PERM = np.concatenate([_old_base + 0, _old_base + 1])

_SMM = np.kron(np.eye(HEADS, dtype=np.float32), np.ones((16, 16), np.float32))

_CORNERS = ((0, 0), (0, 1), (1, 0), (1, 1))


def _stage_a_math(qry, val, refx, refy, Wv, bv, Woffp, boffp, Wattn, battn, consts, smm):
    v = jnp.dot(val, Wv, preferred_element_type=jnp.float32) + bv
    off = jnp.dot(qry, Woffp, preferred_element_type=jnp.float32) + boffp
    logits = jnp.dot(qry, Wattn, preferred_element_type=jnp.float32) + battn
    e = jnp.exp(logits)
    denom = jnp.dot(e, smm, preferred_element_type=jnp.float32)
    aw = e / denom
    wl = consts[0:1, :]
    hl = consts[1:2, :]
    startl = consts[2:3, :]
    px = refx * wl + off[:, :HLP] - 0.5
    py = refy * hl + off[:, HLP:] - 0.5
    x0 = jnp.floor(px)
    y0 = jnp.floor(py)
    fx = px - x0
    fy = py - y0
    idx_parts = []
    w_parts = []
    for cy, cx in _CORNERS:
        xi = x0 + float(cx)
        yi = y0 + float(cy)
        valid = ((xi >= 0.0) & (xi <= wl - 1.0) & (yi >= 0.0) & (yi <= hl - 1.0))
        xc = jnp.clip(xi, 0.0, wl - 1.0)
        yc = jnp.clip(yi, 0.0, hl - 1.0)
        idx_f = startl + yc * wl + xc
        wx = fx if cx == 1 else (1.0 - fx)
        wy = fy if cy == 1 else (1.0 - fy)
        wgt = wx * wy * aw * valid.astype(jnp.float32)
        idx_parts.append(idx_f.astype(jnp.int32))
        w_parts.append(wgt)
    idx = jnp.concatenate(idx_parts, axis=-1)
    w = jnp.concatenate(w_parts, axis=-1)
    return v, idx, w


def _kernel_a_body(qry_ref, val_ref, refx_ref, refy_ref, Wv_ref, bv_ref, Woffp_ref,
                   boffp_ref, Wattn_ref, battn_ref, consts_ref, smm_ref,
                   v_ref, idx_ref, w_ref):
    v, idx, w = _stage_a_math(
        qry_ref[...], val_ref[...], refx_ref[...], refy_ref[...],
        Wv_ref[...], bv_ref[...], Woffp_ref[...], boffp_ref[...],
        Wattn_ref[...], battn_ref[...], consts_ref[...], smm_ref[...])
    v_ref[...] = v
    idx_ref[...] = idx
    w_ref[...] = w


def _kernel_c_body(s_ref, q_ref, Wout_ref, bout_ref, o_ref):
    o_ref[...] = (jnp.dot(s_ref[...], Wout_ref[...], preferred_element_type=jnp.float32)
                  + bout_ref[...] + q_ref[...])


def _sc_gather(table_hbm, idx_hbm, w_hbm, out_hbm, table_v, idx_v, w_v, out_v):
    wid = lax.axis_index("s") * 2 + lax.axis_index("c")
    b = wid // 16
    r = wid - b * 16
    h = r // 2
    half = r - h * 2
    t2 = b * HEADS + h
    t3 = t2 * 2 + half
    pltpu.sync_copy(table_hbm.at[t3], table_v)

    def chunk(ci, carry):
        e0 = ci * (CQ * 64)
        pltpu.sync_copy(idx_hbm.at[t2, pl.ds(e0, CQ * 64)], idx_v)
        pltpu.sync_copy(w_hbm.at[t2, pl.ds(e0, CQ * 64)], w_v)

        def qstep(qi, c2):
            acc = [jnp.zeros((16,), jnp.float32) for _ in range(4)]
            qb = qi * 64
            for g in range(4):
                iv = idx_v[pl.ds(qb + g * 16, 16)]
                wv = w_v[pl.ds(qb + g * 16, 16)]
                for k in range(16):
                    i = iv[k]
                    wgt = wv[k]
                    row = table_v[pl.ds(i * 16, 16)]
                    acc[k % 4] = acc[k % 4] + wgt * row
            out_v[pl.ds(qi * 16, 16)] = (acc[0] + acc[1]) + (acc[2] + acc[3])
            return c2

        lax.fori_loop(0, CQ, qstep, 0)
        pltpu.sync_copy(out_v, out_hbm.at[t3, pl.ds(ci * (CQ * 16), CQ * 16)])
        return carry

    lax.fori_loop(0, NCHUNK, chunk, 0)


def kernel(query, value, reference_points, spatial_shapes, Wv, bv, Woff, boff,
           Wattn, battn, Wout, bout):
    f32 = jnp.float32
    qflat = query.reshape(ROWS, EMBED)
    vflat = value.reshape(ROWS, EMBED)
    rx = jnp.broadcast_to(reference_points[:, :, None, :, None, 0],
                          (BS, NQ, HEADS, LEVELS, POINTS)).reshape(ROWS, HLP)
    ry = jnp.broadcast_to(reference_points[:, :, None, :, None, 1],
                          (BS, NQ, HEADS, LEVELS, POINTS)).reshape(ROWS, HLP)
    Woffp = Woff[:, _OFF_PERM]
    boffp = boff[_OFF_PERM].reshape(1, 2 * HLP)
    consts = jnp.asarray(_CONSTS)
    smm = jnp.asarray(_SMM)

    row_spec = lambda w: pl.BlockSpec((RBLK, w), lambda i: (i, 0))
    full_spec = lambda a, b: pl.BlockSpec((a, b), lambda i: (0, 0))

    v_out, idx_out, w_out = pl.pallas_call(
        _kernel_a_body,
        grid=(NBLK,),
        in_specs=[row_spec(EMBED), row_spec(EMBED), row_spec(HLP), row_spec(HLP),
                  full_spec(EMBED, EMBED), full_spec(1, EMBED),
                  full_spec(EMBED, 2 * HLP), full_spec(1, 2 * HLP),
                  full_spec(EMBED, HLP), full_spec(1, HLP),
                  full_spec(8, HLP), full_spec(HLP, HLP)],
        out_specs=[row_spec(EMBED), row_spec(4 * HLP), row_spec(4 * HLP)],
        out_shape=[jax.ShapeDtypeStruct((ROWS, EMBED), f32),
                   jax.ShapeDtypeStruct((ROWS, 4 * HLP), jnp.int32),
                   jax.ShapeDtypeStruct((ROWS, 4 * HLP), f32)],
    )(qflat, vflat, rx, ry, Wv, bv.reshape(1, EMBED), Woffp, boffp,
      Wattn, battn.reshape(1, HLP), consts, smm)

    table = v_out.reshape(BS, NK, HEADS, 2, 16).transpose(0, 2, 3, 1, 4) \
        .reshape(BS * HEADS * 2, NK * 16)
    idx_hbm = idx_out.reshape(BS, NQ, 4, HEADS, 16).transpose(0, 3, 1, 2, 4) \
        .reshape(BS * HEADS, NQ * 64)
    w_hbm = w_out.reshape(BS, NQ, 4, HEADS, 16).transpose(0, 3, 1, 2, 4) \
        .reshape(BS * HEADS, NQ * 64)

    sc_call = pl.kernel(
        _sc_gather,
        mesh=plsc.VectorSubcoreMesh(core_axis_name="c", subcore_axis_name="s"),
        out_type=jax.ShapeDtypeStruct((BS * HEADS * 2, NQ * 16), f32),
        scratch_types=[pltpu.VMEM((NK * 16,), f32),
                       pltpu.VMEM((CQ * 64,), jnp.int32),
                       pltpu.VMEM((CQ * 64,), f32),
                       pltpu.VMEM((CQ * 16,), f32)],
    )
    out_sc = sc_call(table, idx_hbm, w_hbm)

    sampled = out_sc.reshape(BS, HEADS, 2, NQ, 16).transpose(0, 3, 1, 2, 4) \
        .reshape(ROWS, EMBED)

    final = pl.pallas_call(
        _kernel_c_body,
        grid=(NBLK,),
        in_specs=[row_spec(EMBED), row_spec(EMBED),
                  full_spec(EMBED, EMBED), full_spec(1, EMBED)],
        out_specs=row_spec(EMBED),
        out_shape=jax.ShapeDtypeStruct((ROWS, EMBED), f32),
    )(sampled, qflat, Wout, bout.reshape(1, EMBED))

    return final.reshape(BS, NQ, EMBED)


# speedup vs baseline: 62.2202x; 1.0090x over previous
import functools
import numpy as np
import jax
import jax.numpy as jnp
from jax import lax
from jax.experimental import pallas as pl
from jax.experimental.pallas import tpu as pltpu
from jax.experimental.pallas import tpu_sc as plsc

BS = 2
EMBED = 256
HEADS = 8
LEVELS = 4
POINTS = 4
_SHAPES = np.array([[64, 64], [32, 32], [16, 16], [8, 8]], dtype=np.int64)
_SIZES = (_SHAPES[:, 0] * _SHAPES[:, 1]).astype(np.int64)
NK = int(_SIZES.sum())
NQ = NK
HLP = HEADS * LEVELS * POINTS
ROWS = BS * NQ
RBLK = 544
NBLK = ROWS // RBLK
CQ = 160
NCHUNK = NQ // CQ

_lane_l = np.tile(np.repeat(np.arange(LEVELS), POINTS), HEADS)
_WL = _SHAPES[_lane_l, 1].astype(np.float32)
_HL = _SHAPES[_lane_l, 0].astype(np.float32)
_STARTS = np.concatenate([[0], np.cumsum(_SIZES)]).astype(np.float32)
_STARTL = _STARTS[_lane_l]

_CONSTS = np.zeros((8, HLP), np.float32)
_CONSTS[0] = _WL
_CONSTS[1] = _HL
_CONSTS[2] = _STARTL

_h = np.arange(HLP) // (LEVELS * POINTS)
_l = (np.arange(HLP) // POINTS) % LEVELS
_p = np.arange(HLP) % POINTS
_old_base = ((_h * LEVELS + _l) * POINTS + _p) * 2
_OFF_PERM = np.concatenate([_old_base + 0, _old_base + 1])

_SMM = np.kron(np.eye(HEADS, dtype=np.float32), np.ones((16, 16), np.float32))

_CORNERS = ((0, 0), (0, 1), (1, 0), (1, 1))


def _stage_a_math(qry, val, refx, refy, Wv, bv, Woffp, boffp, Wattn, battn, consts, smm):
    v = jnp.dot(val, Wv, preferred_element_type=jnp.float32) + bv
    off = jnp.dot(qry, Woffp, preferred_element_type=jnp.float32) + boffp
    logits = jnp.dot(qry, Wattn, preferred_element_type=jnp.float32) + battn
    e = jnp.exp(logits)
    denom = jnp.dot(e, smm, preferred_element_type=jnp.float32)
    aw = e / denom
    wl = consts[0:1, :]
    hl = consts[1:2, :]
    startl = consts[2:3, :]
    px = refx * wl + off[:, :HLP] - 0.5
    py = refy * hl + off[:, HLP:] - 0.5
    x0 = jnp.floor(px)
    y0 = jnp.floor(py)
    fx = px - x0
    fy = py - y0
    idx_parts = []
    w_parts = []
    for cy, cx in _CORNERS:
        xi = x0 + float(cx)
        yi = y0 + float(cy)
        valid = ((xi >= 0.0) & (xi <= wl - 1.0) & (yi >= 0.0) & (yi <= hl - 1.0))
        xc = jnp.clip(xi, 0.0, wl - 1.0)
        yc = jnp.clip(yi, 0.0, hl - 1.0)
        idx_f = (startl + yc * wl + xc) * 16.0
        wx = fx if cx == 1 else (1.0 - fx)
        wy = fy if cy == 1 else (1.0 - fy)
        wgt = wx * wy * aw * valid.astype(jnp.float32)
        idx_parts.append(idx_f.astype(jnp.int32))
        w_parts.append(wgt)
    idx = jnp.concatenate(idx_parts, axis=-1)
    w = jnp.concatenate(w_parts, axis=-1)
    return v, idx, w


def _kernel_a_body(qry_ref, val_ref, refx_ref, refy_ref, Wv_ref, bv_ref, Woffp_ref,
                   boffp_ref, Wattn_ref, battn_ref, consts_ref, smm_ref,
                   v_ref, idx_ref, w_ref):
    v, idx, w = _stage_a_math(
        qry_ref[...], val_ref[...], refx_ref[...], refy_ref[...],
        Wv_ref[...], bv_ref[...], Woffp_ref[...], boffp_ref[...],
        Wattn_ref[...], battn_ref[...], consts_ref[...], smm_ref[...])
    v_ref[...] = v
    idx_ref[...] = idx
    w_ref[...] = w


def _kernel_c_body(s_ref, q_ref, Wout_ref, bout_ref, o_ref):
    o_ref[...] = (jnp.dot(s_ref[...], Wout_ref[...], preferred_element_type=jnp.float32)
                  + bout_ref[...] + q_ref[...])


def _sc_gather(table_hbm, idx_hbm, w_hbm, out_hbm, table_v, idx_v, w_v, out_v):
    wid = lax.axis_index("s") * 2 + lax.axis_index("c")
    b = wid // 16
    r = wid - b * 16
    h = r // 2
    half = r - h * 2
    t2 = b * HEADS + h
    t3 = t2 * 2 + half
    pltpu.sync_copy(table_hbm.at[t3], table_v)

    def chunk(ci, carry):
        e0 = ci * (CQ * 64)
        pltpu.sync_copy(idx_hbm.at[t2, pl.ds(e0, CQ * 64)], idx_v)
        pltpu.sync_copy(w_hbm.at[t2, pl.ds(e0, CQ * 64)], w_v)

        def qstep(qi, c2):
            acc = [jnp.zeros((16,), jnp.float32) for _ in range(4)]
            qb = qi * 64
            for g in range(4):
                iv = idx_v[pl.ds(qb + g * 16, 16)]
                wv = w_v[pl.ds(qb + g * 16, 16)]
                for k in range(16):
                    i = iv[k]
                    wgt = wv[k]
                    row = table_v[pl.ds(i, 16)]
                    acc[k % 4] = acc[k % 4] + wgt * row
            out_v[pl.ds(qi * 16, 16)] = (acc[0] + acc[1]) + (acc[2] + acc[3])
            return c2

        lax.fori_loop(0, CQ, qstep, 0)
        pltpu.sync_copy(out_v, out_hbm.at[t3, pl.ds(ci * (CQ * 16), CQ * 16)])
        return carry

    lax.fori_loop(0, NCHUNK, chunk, 0)


def kernel(query, value, reference_points, spatial_shapes, Wv, bv, Woff, boff,
           Wattn, battn, Wout, bout):
    f32 = jnp.float32
    qflat = query.reshape(ROWS, EMBED)
    vflat = value.reshape(ROWS, EMBED)
    rx = jnp.broadcast_to(reference_points[:, :, None, :, None, 0],
                          (BS, NQ, HEADS, LEVELS, POINTS)).reshape(ROWS, HLP)
    ry = jnp.broadcast_to(reference_points[:, :, None, :, None, 1],
                          (BS, NQ, HEADS, LEVELS, POINTS)).reshape(ROWS, HLP)
    Woffp = Woff[:, _OFF_PERM]
    boffp = boff[_OFF_PERM].reshape(1, 2 * HLP)
    consts = jnp.asarray(_CONSTS)
    smm = jnp.asarray(_SMM)

    row_spec = lambda w: pl.BlockSpec((RBLK, w), lambda i: (i, 0))
    full_spec = lambda a, b: pl.BlockSpec((a, b), lambda i: (0, 0))

    v_out, idx_out, w_out = pl.pallas_call(
        _kernel_a_body,
        grid=(NBLK,),
        in_specs=[row_spec(EMBED), row_spec(EMBED), row_spec(HLP), row_spec(HLP),
                  full_spec(EMBED, EMBED), full_spec(1, EMBED),
                  full_spec(EMBED, 2 * HLP), full_spec(1, 2 * HLP),
                  full_spec(EMBED, HLP), full_spec(1, HLP),
                  full_spec(8, HLP), full_spec(HLP, HLP)],
        out_specs=[row_spec(EMBED), row_spec(4 * HLP), row_spec(4 * HLP)],
        out_shape=[jax.ShapeDtypeStruct((ROWS, EMBED), f32),
                   jax.ShapeDtypeStruct((ROWS, 4 * HLP), jnp.int32),
                   jax.ShapeDtypeStruct((ROWS, 4 * HLP), f32)],
    )(qflat, vflat, rx, ry, Wv, bv.reshape(1, EMBED), Woffp, boffp,
      Wattn, battn.reshape(1, HLP), consts, smm)

    table = v_out.reshape(BS, NK, HEADS, 2, 16).transpose(0, 2, 3, 1, 4) \
        .reshape(BS * HEADS * 2, NK * 16)
    idx_hbm = idx_out.reshape(BS, NQ, 4, HEADS, 16).transpose(0, 3, 1, 2, 4) \
        .reshape(BS * HEADS, NQ * 64)
    w_hbm = w_out.reshape(BS, NQ, 4, HEADS, 16).transpose(0, 3, 1, 2, 4) \
        .reshape(BS * HEADS, NQ * 64)

    sc_call = pl.kernel(
        _sc_gather,
        mesh=plsc.VectorSubcoreMesh(core_axis_name="c", subcore_axis_name="s"),
        out_type=jax.ShapeDtypeStruct((BS * HEADS * 2, NQ * 16), f32),
        scratch_types=[pltpu.VMEM((NK * 16,), f32),
                       pltpu.VMEM((CQ * 64,), jnp.int32),
                       pltpu.VMEM((CQ * 64,), f32),
                       pltpu.VMEM((CQ * 16,), f32)],
    )
    out_sc = sc_call(table, idx_hbm, w_hbm)

    sampled = out_sc.reshape(BS, HEADS, 2, NQ, 16).transpose(0, 3, 1, 2, 4) \
        .reshape(ROWS, EMBED)

    final = pl.pallas_call(
        _kernel_c_body,
        grid=(NBLK,),
        in_specs=[row_spec(EMBED), row_spec(EMBED),
                  full_spec(EMBED, EMBED), full_spec(1, EMBED)],
        out_specs=row_spec(EMBED),
        out_shape=jax.ShapeDtypeStruct((ROWS, EMBED), f32),
    )(sampled, qflat, Wout, bout.reshape(1, EMBED))

    return final.reshape(BS, NQ, EMBED)


# speedup vs baseline: 90.1397x; 1.4487x over previous
import functools
import numpy as np
import jax
import jax.numpy as jnp
from jax import lax
from jax.experimental import pallas as pl
from jax.experimental.pallas import tpu as pltpu
from jax.experimental.pallas import tpu_sc as plsc

BS = 2
EMBED = 256
HEADS = 8
LEVELS = 4
POINTS = 4
_SHAPES = np.array([[64, 64], [32, 32], [16, 16], [8, 8]], dtype=np.int64)
_SIZES = (_SHAPES[:, 0] * _SHAPES[:, 1]).astype(np.int64)
NK = int(_SIZES.sum())
NQ = NK
HLP = HEADS * LEVELS * POINTS
ROWS = BS * NQ
RBLK = 544
NBLK = ROWS // RBLK
CQ = 160
NCHUNK = NQ // CQ

_lane_l = np.tile(np.repeat(np.arange(LEVELS), POINTS), HEADS)
_WL = _SHAPES[_lane_l, 1].astype(np.float32)
_HL = _SHAPES[_lane_l, 0].astype(np.float32)
_STARTS = np.concatenate([[0], np.cumsum(_SIZES)]).astype(np.float32)
_STARTL = _STARTS[_lane_l]

_CONSTS = np.zeros((8, HLP), np.float32)
_CONSTS[0] = _WL
_CONSTS[1] = _HL
_CONSTS[2] = _STARTL

_h = np.arange(HLP) // (LEVELS * POINTS)
_l = (np.arange(HLP) // POINTS) % LEVELS
_p = np.arange(HLP) % POINTS
_old_base = ((_h * LEVELS + _l) * POINTS + _p) * 2
_OFF_PERM = np.concatenate([_old_base + 0, _old_base + 1])

_SMM = np.kron(np.eye(HEADS, dtype=np.float32), np.ones((16, 16), np.float32))

_CORNERS = ((0, 0), (0, 1), (1, 0), (1, 1))


def _stage_a_math(qry, val, refx, refy, Wv, bv, Woffp, boffp, Wattn, battn, consts, smm):
    v = jnp.dot(val, Wv, preferred_element_type=jnp.float32) + bv
    off = jnp.dot(qry, Woffp, preferred_element_type=jnp.float32) + boffp
    logits = jnp.dot(qry, Wattn, preferred_element_type=jnp.float32) + battn
    e = jnp.exp(logits)
    denom = jnp.dot(e, smm, preferred_element_type=jnp.float32)
    aw = e / denom
    wl = consts[0:1, :]
    hl = consts[1:2, :]
    startl = consts[2:3, :]
    px = refx * wl + off[:, :HLP] - 0.5
    py = refy * hl + off[:, HLP:] - 0.5
    x0 = jnp.floor(px)
    y0 = jnp.floor(py)
    fx = px - x0
    fy = py - y0
    idx_parts = []
    w_parts = []
    for cy, cx in _CORNERS:
        xi = x0 + float(cx)
        yi = y0 + float(cy)
        valid = ((xi >= 0.0) & (xi <= wl - 1.0) & (yi >= 0.0) & (yi <= hl - 1.0))
        xc = jnp.clip(xi, 0.0, wl - 1.0)
        yc = jnp.clip(yi, 0.0, hl - 1.0)
        idx_f = (startl + yc * wl + xc) * 16.0
        wx = fx if cx == 1 else (1.0 - fx)
        wy = fy if cy == 1 else (1.0 - fy)
        wgt = wx * wy * aw * valid.astype(jnp.float32)
        idx_parts.append(idx_f.astype(jnp.int32))
        w_parts.append(wgt)
    idx = jnp.concatenate(idx_parts, axis=-1)
    w = jnp.concatenate(w_parts, axis=-1)
    return v, idx, w


def _head_slice(x, h):
    return jnp.concatenate([x[:, co * HLP + h * 16: co * HLP + h * 16 + 16]
                            for co in range(4)], axis=-1)


def _kernel_a_body(qry_ref, val_ref, refx_ref, refy_ref, Wv_ref, bv_ref, Woffp_ref,
                   boffp_ref, Wattn_ref, battn_ref, consts_ref, smm_ref,
                   v_ref, idx_ref, w_ref):
    v, idx, w = _stage_a_math(
        qry_ref[...], val_ref[...], refx_ref[...], refy_ref[...],
        Wv_ref[...], bv_ref[...], Woffp_ref[...], boffp_ref[...],
        Wattn_ref[...], battn_ref[...], consts_ref[...], smm_ref[...])
    v_ref[...] = v
    for h in range(HEADS):
        idx_ref[0, h] = _head_slice(idx, h)
        w_ref[0, h] = _head_slice(w, h)


def _kernel_c_body(s_ref, q_ref, Wout_ref, bout_ref, o_ref):
    s = jnp.concatenate([s_ref[0, h, half] for h in range(HEADS)
                         for half in range(2)], axis=-1)
    o_ref[...] = (jnp.dot(s, Wout_ref[...], preferred_element_type=jnp.float32)
                  + bout_ref[...] + q_ref[...])


def _sc_gather(table_hbm, idx_hbm, w_hbm, out_hbm, table_v, idx_v, w_v, out_v):
    wid = lax.axis_index("s") * 2 + lax.axis_index("c")
    b = wid // 16
    r = wid - b * 16
    h = r // 2
    half = r - h * 2
    t2 = b * HEADS + h
    t3 = t2 * 2 + half
    pltpu.sync_copy(table_hbm.at[t3], table_v)

    def chunk(ci, carry):
        e0 = ci * (CQ * 64)
        pltpu.sync_copy(idx_hbm.at[t2, pl.ds(e0, CQ * 64)], idx_v)
        pltpu.sync_copy(w_hbm.at[t2, pl.ds(e0, CQ * 64)], w_v)

        def qstep(qi, c2):
            acc = [jnp.zeros((16,), jnp.float32) for _ in range(4)]
            qb = qi * 64
            for g in range(4):
                iv = idx_v[pl.ds(qb + g * 16, 16)]
                wv = w_v[pl.ds(qb + g * 16, 16)]
                for k in range(16):
                    i = iv[k]
                    wgt = wv[k]
                    row = table_v[pl.ds(i, 16)]
                    acc[k % 4] = acc[k % 4] + wgt * row
            out_v[pl.ds(qi * 16, 16)] = (acc[0] + acc[1]) + (acc[2] + acc[3])
            return c2

        lax.fori_loop(0, CQ, qstep, 0)
        pltpu.sync_copy(out_v, out_hbm.at[t3, pl.ds(ci * (CQ * 16), CQ * 16)])
        return carry

    lax.fori_loop(0, NCHUNK, chunk, 0)


def kernel(query, value, reference_points, spatial_shapes, Wv, bv, Woff, boff,
           Wattn, battn, Wout, bout):
    f32 = jnp.float32
    qflat = query.reshape(ROWS, EMBED)
    vflat = value.reshape(ROWS, EMBED)
    rx = jnp.broadcast_to(reference_points[:, :, None, :, None, 0],
                          (BS, NQ, HEADS, LEVELS, POINTS)).reshape(ROWS, HLP)
    ry = jnp.broadcast_to(reference_points[:, :, None, :, None, 1],
                          (BS, NQ, HEADS, LEVELS, POINTS)).reshape(ROWS, HLP)
    Woffp = Woff[:, _OFF_PERM]
    boffp = boff[_OFF_PERM].reshape(1, 2 * HLP)
    consts = jnp.asarray(_CONSTS)
    smm = jnp.asarray(_SMM)

    row_spec = lambda w: pl.BlockSpec((RBLK, w), lambda i: (i, 0))
    full_spec = lambda a, b: pl.BlockSpec((a, b), lambda i: (0, 0))

    v_out, idx_out, w_out = pl.pallas_call(
        _kernel_a_body,
        grid=(NBLK,),
        in_specs=[row_spec(EMBED), row_spec(EMBED), row_spec(HLP), row_spec(HLP),
                  full_spec(EMBED, EMBED), full_spec(1, EMBED),
                  full_spec(EMBED, 2 * HLP), full_spec(1, 2 * HLP),
                  full_spec(EMBED, HLP), full_spec(1, HLP),
                  full_spec(8, HLP), full_spec(HLP, HLP)],
        out_specs=[row_spec(EMBED),
                   pl.BlockSpec((1, HEADS, RBLK, 64),
                                lambda i: (i // (NQ // RBLK), 0, i % (NQ // RBLK), 0)),
                   pl.BlockSpec((1, HEADS, RBLK, 64),
                                lambda i: (i // (NQ // RBLK), 0, i % (NQ // RBLK), 0))],
        out_shape=[jax.ShapeDtypeStruct((ROWS, EMBED), f32),
                   jax.ShapeDtypeStruct((BS, HEADS, NQ, 64), jnp.int32),
                   jax.ShapeDtypeStruct((BS, HEADS, NQ, 64), f32)],
    )(qflat, vflat, rx, ry, Wv, bv.reshape(1, EMBED), Woffp, boffp,
      Wattn, battn.reshape(1, HLP), consts, smm)

    table = v_out.reshape(BS, NK, HEADS, 2, 16).transpose(0, 2, 3, 1, 4) \
        .reshape(BS * HEADS * 2, NK * 16)
    idx_hbm = idx_out.reshape(BS * HEADS, NQ * 64)
    w_hbm = w_out.reshape(BS * HEADS, NQ * 64)

    sc_call = pl.kernel(
        _sc_gather,
        mesh=plsc.VectorSubcoreMesh(core_axis_name="c", subcore_axis_name="s"),
        out_type=jax.ShapeDtypeStruct((BS * HEADS * 2, NQ * 16), f32),
        scratch_types=[pltpu.VMEM((NK * 16,), f32),
                       pltpu.VMEM((CQ * 64,), jnp.int32),
                       pltpu.VMEM((CQ * 64,), f32),
                       pltpu.VMEM((CQ * 16,), f32)],
    )
    out_sc = sc_call(table, idx_hbm, w_hbm)

    sampled5 = out_sc.reshape(BS, HEADS, 2, NQ, 16)

    final = pl.pallas_call(
        _kernel_c_body,
        grid=(NBLK,),
        in_specs=[pl.BlockSpec((1, HEADS, 2, RBLK, 16),
                               lambda i: (i // (NQ // RBLK), 0, 0, i % (NQ // RBLK), 0)),
                  row_spec(EMBED),
                  full_spec(EMBED, EMBED), full_spec(1, EMBED)],
        out_specs=row_spec(EMBED),
        out_shape=jax.ShapeDtypeStruct((ROWS, EMBED), f32),
    )(sampled5, qflat, Wout, bout.reshape(1, EMBED))

    return final.reshape(BS, NQ, EMBED)


# speedup vs baseline: 91.6317x; 1.0166x over previous
import functools
import numpy as np
import jax
import jax.numpy as jnp
from jax import lax
from jax.experimental import pallas as pl
from jax.experimental.pallas import tpu as pltpu
from jax.experimental.pallas import tpu_sc as plsc

BS = 2
EMBED = 256
HEADS = 8
LEVELS = 4
POINTS = 4
_SHAPES = np.array([[64, 64], [32, 32], [16, 16], [8, 8]], dtype=np.int64)
_SIZES = (_SHAPES[:, 0] * _SHAPES[:, 1]).astype(np.int64)
NK = int(_SIZES.sum())
NQ = NK
HLP = HEADS * LEVELS * POINTS
ROWS = BS * NQ
RBLK = 544
NBLK = ROWS // RBLK
CQ = 160
NCHUNK = NQ // CQ

_lane_l = np.tile(np.repeat(np.arange(LEVELS), POINTS), HEADS)
_WL = _SHAPES[_lane_l, 1].astype(np.float32)
_HL = _SHAPES[_lane_l, 0].astype(np.float32)
_STARTS = np.concatenate([[0], np.cumsum(_SIZES)]).astype(np.float32)
_STARTL = _STARTS[_lane_l]

_CONSTS = np.zeros((8, HLP), np.float32)
_CONSTS[0] = _WL
_CONSTS[1] = _HL
_CONSTS[2] = _STARTL

_h = np.arange(HLP) // (LEVELS * POINTS)
_l = (np.arange(HLP) // POINTS) % LEVELS
_p = np.arange(HLP) % POINTS
_old_base = ((_h * LEVELS + _l) * POINTS + _p) * 2
_OFF_PERM = np.concatenate([_old_base + 0, _old_base + 1])

_SMM = np.kron(np.eye(HEADS, dtype=np.float32), np.ones((16, 16), np.float32))

_CORNERS = ((0, 0), (0, 1), (1, 0), (1, 1))


def _stage_a_math(qry, val, refx, refy, Wv, bv, Woffp, boffp, Wattn, battn, consts, smm):
    v = jnp.dot(val, Wv, preferred_element_type=jnp.float32) + bv
    off = jnp.dot(qry, Woffp, preferred_element_type=jnp.float32) + boffp
    logits = jnp.dot(qry, Wattn, preferred_element_type=jnp.float32) + battn
    e = jnp.exp(logits)
    denom = jnp.dot(e, smm, preferred_element_type=jnp.float32)
    aw = e / denom
    wl = consts[0:1, :]
    hl = consts[1:2, :]
    startl = consts[2:3, :]
    px = refx * wl + off[:, :HLP] - 0.5
    py = refy * hl + off[:, HLP:] - 0.5
    x0 = jnp.floor(px)
    y0 = jnp.floor(py)
    fx = px - x0
    fy = py - y0
    idx_parts = []
    w_parts = []
    for cy, cx in _CORNERS:
        xi = x0 + float(cx)
        yi = y0 + float(cy)
        valid = ((xi >= 0.0) & (xi <= wl - 1.0) & (yi >= 0.0) & (yi <= hl - 1.0))
        xc = jnp.clip(xi, 0.0, wl - 1.0)
        yc = jnp.clip(yi, 0.0, hl - 1.0)
        idx_f = (startl + yc * wl + xc) * 16.0
        wx = fx if cx == 1 else (1.0 - fx)
        wy = fy if cy == 1 else (1.0 - fy)
        wgt = wx * wy * aw * valid.astype(jnp.float32)
        idx_parts.append(idx_f.astype(jnp.int32))
        w_parts.append(wgt)
    idx = jnp.concatenate(idx_parts, axis=-1)
    w = jnp.concatenate(w_parts, axis=-1)
    return v, idx, w


def _head_slice(x, h):
    return jnp.concatenate([x[:, co * HLP + h * 16: co * HLP + h * 16 + 16]
                            for co in range(4)], axis=-1)


def _kernel_a_body(qry_ref, val_ref, refx_ref, refy_ref, Wv_ref, bv_ref, Woffp_ref,
                   boffp_ref, Wattn_ref, battn_ref, consts_ref, smm_ref,
                   v_ref, idx_ref, w_ref):
    v, idx, w = _stage_a_math(
        qry_ref[...], val_ref[...], refx_ref[...], refy_ref[...],
        Wv_ref[...], bv_ref[...], Woffp_ref[...], boffp_ref[...],
        Wattn_ref[...], battn_ref[...], consts_ref[...], smm_ref[...])
    for h in range(HEADS):
        for half in range(2):
            v_ref[0, h, half] = v[:, h * 32 + half * 16: h * 32 + half * 16 + 16]
        idx_ref[0, h] = _head_slice(idx, h)
        w_ref[0, h] = _head_slice(w, h)


def _kernel_c_body(s_ref, q_ref, Wout_ref, bout_ref, o_ref):
    s = jnp.concatenate([s_ref[0, h, half] for h in range(HEADS)
                         for half in range(2)], axis=-1)
    o_ref[...] = (jnp.dot(s, Wout_ref[...], preferred_element_type=jnp.float32)
                  + bout_ref[...] + q_ref[...])


def _sc_gather(table_hbm, idx_hbm, w_hbm, out_hbm, table_v, idx_v, w_v, out_v):
    wid = lax.axis_index("s") * 2 + lax.axis_index("c")
    b = wid // 16
    r = wid - b * 16
    h = r // 2
    half = r - h * 2
    t2 = b * HEADS + h
    t3 = t2 * 2 + half
    pltpu.sync_copy(table_hbm.at[t3], table_v)

    def chunk(ci, carry):
        e0 = ci * (CQ * 64)
        pltpu.sync_copy(idx_hbm.at[t2, pl.ds(e0, CQ * 64)], idx_v)
        pltpu.sync_copy(w_hbm.at[t2, pl.ds(e0, CQ * 64)], w_v)

        def qstep(qi, c2):
            acc = [jnp.zeros((16,), jnp.float32) for _ in range(4)]
            qb = qi * 64
            for g in range(4):
                iv = idx_v[pl.ds(qb + g * 16, 16)]
                wv = w_v[pl.ds(qb + g * 16, 16)]
                for k in range(16):
                    i = iv[k]
                    wgt = wv[k]
                    row = table_v[pl.ds(i, 16)]
                    acc[k % 4] = acc[k % 4] + wgt * row
            out_v[pl.ds(qi * 16, 16)] = (acc[0] + acc[1]) + (acc[2] + acc[3])
            return c2

        lax.fori_loop(0, CQ, qstep, 0)
        pltpu.sync_copy(out_v, out_hbm.at[t3, pl.ds(ci * (CQ * 16), CQ * 16)])
        return carry

    lax.fori_loop(0, NCHUNK, chunk, 0)


def kernel(query, value, reference_points, spatial_shapes, Wv, bv, Woff, boff,
           Wattn, battn, Wout, bout):
    f32 = jnp.float32
    qflat = query.reshape(ROWS, EMBED)
    vflat = value.reshape(ROWS, EMBED)
    rx = jnp.broadcast_to(reference_points[:, :, None, :, None, 0],
                          (BS, NQ, HEADS, LEVELS, POINTS)).reshape(ROWS, HLP)
    ry = jnp.broadcast_to(reference_points[:, :, None, :, None, 1],
                          (BS, NQ, HEADS, LEVELS, POINTS)).reshape(ROWS, HLP)
    Woffp = Woff[:, _OFF_PERM]
    boffp = boff[_OFF_PERM].reshape(1, 2 * HLP)
    consts = jnp.asarray(_CONSTS)
    smm = jnp.asarray(_SMM)

    row_spec = lambda w: pl.BlockSpec((RBLK, w), lambda i: (i, 0))
    full_spec = lambda a, b: pl.BlockSpec((a, b), lambda i: (0, 0))

    v_out, idx_out, w_out = pl.pallas_call(
        _kernel_a_body,
        grid=(NBLK,),
        in_specs=[row_spec(EMBED), row_spec(EMBED), row_spec(HLP), row_spec(HLP),
                  full_spec(EMBED, EMBED), full_spec(1, EMBED),
                  full_spec(EMBED, 2 * HLP), full_spec(1, 2 * HLP),
                  full_spec(EMBED, HLP), full_spec(1, HLP),
                  full_spec(8, HLP), full_spec(HLP, HLP)],
        out_specs=[pl.BlockSpec((1, HEADS, 2, RBLK, 16),
                                lambda i: (i // (NQ // RBLK), 0, 0, i % (NQ // RBLK), 0)),
                   pl.BlockSpec((1, HEADS, RBLK, 64),
                                lambda i: (i // (NQ // RBLK), 0, i % (NQ // RBLK), 0)),
                   pl.BlockSpec((1, HEADS, RBLK, 64),
                                lambda i: (i // (NQ // RBLK), 0, i % (NQ // RBLK), 0))],
        out_shape=[jax.ShapeDtypeStruct((BS, HEADS, 2, NK, 16), f32),
                   jax.ShapeDtypeStruct((BS, HEADS, NQ, 64), jnp.int32),
                   jax.ShapeDtypeStruct((BS, HEADS, NQ, 64), f32)],
    )(qflat, vflat, rx, ry, Wv, bv.reshape(1, EMBED), Woffp, boffp,
      Wattn, battn.reshape(1, HLP), consts, smm)

    table = v_out.reshape(BS * HEADS * 2, NK * 16)
    idx_hbm = idx_out.reshape(BS * HEADS, NQ * 64)
    w_hbm = w_out.reshape(BS * HEADS, NQ * 64)

    sc_call = pl.kernel(
        _sc_gather,
        mesh=plsc.VectorSubcoreMesh(core_axis_name="c", subcore_axis_name="s"),
        out_type=jax.ShapeDtypeStruct((BS * HEADS * 2, NQ * 16), f32),
        scratch_types=[pltpu.VMEM((NK * 16,), f32),
                       pltpu.VMEM((CQ * 64,), jnp.int32),
                       pltpu.VMEM((CQ * 64,), f32),
                       pltpu.VMEM((CQ * 16,), f32)],
    )
    out_sc = sc_call(table, idx_hbm, w_hbm)

    sampled5 = out_sc.reshape(BS, HEADS, 2, NQ, 16)

    final = pl.pallas_call(
        _kernel_c_body,
        grid=(NBLK,),
        in_specs=[pl.BlockSpec((1, HEADS, 2, RBLK, 16),
                               lambda i: (i // (NQ // RBLK), 0, 0, i % (NQ // RBLK), 0)),
                  row_spec(EMBED),
                  full_spec(EMBED, EMBED), full_spec(1, EMBED)],
        out_specs=row_spec(EMBED),
        out_shape=jax.ShapeDtypeStruct((ROWS, EMBED), f32),
    )(sampled5, qflat, Wout, bout.reshape(1, EMBED))

    return final.reshape(BS, NQ, EMBED)
